# Initial kernel scaffold; baseline (speedup 1.0000x reference)
import jax
import jax.numpy as jnp
from jax.experimental import pallas as pl


def kernel(z_real, z_imag, embedding):
    raise NotImplementedError("write your pallas kernel here")


# speedup vs baseline: 1.0875x; 1.0875x over previous
import functools

import jax
import jax.numpy as jnp
from jax import lax
from jax.experimental import pallas as pl
from jax.experimental.pallas import tpu as pltpu
from jax.experimental.pallas import tpu_sc as plsc

_M = 16 * 1024
_K = 8192
_D = 256

_BM = 1024
_BN = 2048
_MT = _M // _BM
_NT = _K // _BN

_NC = 2
_NS = 16
_NW = _NC * _NS
_ROWS_PER_W = _M // _NW
_CHUNK = 256


def _vq_tc_body(x_ref, a_ref, e_ref, b_ref, idx_ref, val_ref, minv, mini):
    n = pl.program_id(1)
    scores = lax.dot_general(
        x_ref[...], e_ref[...],
        dimension_numbers=(((1,), (1,)), ((), ())),
        preferred_element_type=jnp.float32,
    )
    dist = (a_ref[...][:, None] - 2.0 * scores) + b_ref[...][None, :]
    lmin = jnp.min(dist, axis=1)
    cols = lax.broadcasted_iota(jnp.int32, dist.shape, 1)
    lidx = jnp.min(jnp.where(dist == lmin[:, None], cols, _K), axis=1) + n * _BN

    @pl.when(n == 0)
    def _():
        minv[...] = lmin
        mini[...] = lidx

    @pl.when(n > 0)
    def _():
        better = lmin < minv[...]
        minv[...] = jnp.where(better, lmin, minv[...])
        mini[...] = jnp.where(better, lidx, mini[...])

    @pl.when(n == _NT - 1)
    def _():
        idx_ref[...] = mini[...]
        val_ref[...] = minv[...]


_vq_tc_call = pl.pallas_call(
    _vq_tc_body,
    grid=(_MT, _NT),
    in_specs=[
        pl.BlockSpec((_BM, _D), lambda m, n: (m, 0)),
        pl.BlockSpec((_BM,), lambda m, n: (m,)),
        pl.BlockSpec((_BN, _D), lambda m, n: (n, 0)),
        pl.BlockSpec((_BN,), lambda m, n: (n,)),
    ],
    out_specs=[
        pl.BlockSpec((_BM,), lambda m, n: (m,)),
        pl.BlockSpec((_BM,), lambda m, n: (m,)),
    ],
    out_shape=[
        jax.ShapeDtypeStruct((_M,), jnp.int32),
        jax.ShapeDtypeStruct((_M,), jnp.float32),
    ],
    scratch_shapes=[
        pltpu.VMEM((_BM,), jnp.float32),
        pltpu.VMEM((_BM,), jnp.int32),
    ],
)


def _sc_gather_body(table_hbm, idx_hbm, out_hbm, idx_v, rows_v, sem):
    wid = lax.axis_index("s") * _NC + lax.axis_index("c")
    base = wid * _ROWS_PER_W
    pltpu.sync_copy(idx_hbm.at[pl.ds(base, _ROWS_PER_W)], idx_v)
    for c in range(_ROWS_PER_W // _CHUNK):
        pltpu.async_copy(
            table_hbm.at[idx_v.at[pl.ds(c * _CHUNK, _CHUNK)]], rows_v, sem
        ).wait()
        pltpu.sync_copy(rows_v, out_hbm.at[pl.ds(base + c * _CHUNK, _CHUNK)])


@functools.cache
def _sc_gather():
    return pl.kernel(
        _sc_gather_body,
        out_type=jax.ShapeDtypeStruct((_M, _D), jnp.float32),
        mesh=plsc.VectorSubcoreMesh(core_axis_name="c", subcore_axis_name="s"),
        scratch_types=[
            pltpu.VMEM((_ROWS_PER_W,), jnp.int32),
            pltpu.VMEM((_CHUNK, _D), jnp.float32),
            pltpu.SemaphoreType.DMA,
        ],
    )


def kernel(z_real, z_imag, embedding):
    B, L, D = z_real.shape
    flat = z_real.reshape(B * L, D)
    a = jnp.sum(flat ** 2, axis=1)
    b = jnp.sum(embedding ** 2, axis=1)
    idx, minval = _vq_tc_call(flat, a, embedding, b)
    zq = _sc_gather()(embedding, idx).reshape(B, L, D)
    vq_loss = jnp.sum(minval) * (1.25 / (B * L * D))
    return zq, z_imag, vq_loss


# speedup vs baseline: 1.1514x; 1.0588x over previous
import functools

import jax
import jax.numpy as jnp
from jax import lax
from jax.experimental import pallas as pl
from jax.experimental.pallas import tpu as pltpu
from jax.experimental.pallas import tpu_sc as plsc

_M = 16 * 1024
_K = 8192
_D = 256

_BM = 2048
_BN = 2048
_MT = _M // _BM
_NT = _K // _BN

_NC = 2
_NS = 16
_NW = _NC * _NS
_ROWS_PER_W = _M // _NW
_CHUNK = 256


def _vq_tc_body(x_ref, a_ref, e_ref, b_ref, idx_ref, val_ref, minv, mini):
    n = pl.program_id(1)
    scores = lax.dot_general(
        x_ref[...], e_ref[...],
        dimension_numbers=(((1,), (1,)), ((), ())),
        preferred_element_type=jnp.float32,
    )
    dist = (a_ref[...][:, None] - 2.0 * scores) + b_ref[...][None, :]
    lmin = jnp.min(dist, axis=1)
    cols = lax.broadcasted_iota(jnp.int32, dist.shape, 1)
    lidx = jnp.min(jnp.where(dist == lmin[:, None], cols, _K), axis=1) + n * _BN

    @pl.when(n == 0)
    def _():
        minv[...] = lmin
        mini[...] = lidx

    @pl.when(n > 0)
    def _():
        better = lmin < minv[...]
        minv[...] = jnp.where(better, lmin, minv[...])
        mini[...] = jnp.where(better, lidx, mini[...])

    @pl.when(n == _NT - 1)
    def _():
        idx_ref[...] = mini[...]
        val_ref[...] = minv[...]


_vq_tc_call = pl.pallas_call(
    _vq_tc_body,
    grid=(_MT, _NT),
    in_specs=[
        pl.BlockSpec((_BM, _D), lambda m, n: (m, 0)),
        pl.BlockSpec((_BM,), lambda m, n: (m,)),
        pl.BlockSpec((_BN, _D), lambda m, n: (n, 0)),
        pl.BlockSpec((_BN,), lambda m, n: (n,)),
    ],
    out_specs=[
        pl.BlockSpec((_BM,), lambda m, n: (m,)),
        pl.BlockSpec((_BM,), lambda m, n: (m,)),
    ],
    out_shape=[
        jax.ShapeDtypeStruct((_M,), jnp.int32),
        jax.ShapeDtypeStruct((_M,), jnp.float32),
    ],
    scratch_shapes=[
        pltpu.VMEM((_BM,), jnp.float32),
        pltpu.VMEM((_BM,), jnp.int32),
    ],
)


def _sc_gather_body(table_hbm, idx_hbm, out_hbm, idx_v, rows_v, sem):
    wid = lax.axis_index("s") * _NC + lax.axis_index("c")
    base = wid * _ROWS_PER_W
    pltpu.sync_copy(idx_hbm.at[pl.ds(base, _ROWS_PER_W)], idx_v)
    for c in range(_ROWS_PER_W // _CHUNK):
        pltpu.async_copy(
            table_hbm.at[idx_v.at[pl.ds(c * _CHUNK, _CHUNK)]], rows_v, sem
        ).wait()
        pltpu.sync_copy(rows_v, out_hbm.at[pl.ds(base + c * _CHUNK, _CHUNK)])


@functools.cache
def _sc_gather():
    return pl.kernel(
        _sc_gather_body,
        out_type=jax.ShapeDtypeStruct((_M, _D), jnp.float32),
        mesh=plsc.VectorSubcoreMesh(core_axis_name="c", subcore_axis_name="s"),
        scratch_types=[
            pltpu.VMEM((_ROWS_PER_W,), jnp.int32),
            pltpu.VMEM((_CHUNK, _D), jnp.float32),
            pltpu.SemaphoreType.DMA,
        ],
    )


def kernel(z_real, z_imag, embedding):
    B, L, D = z_real.shape
    flat = z_real.reshape(B * L, D)
    a = jnp.sum(flat ** 2, axis=1)
    b = jnp.sum(embedding ** 2, axis=1)
    idx, minval = _vq_tc_call(flat, a, embedding, b)
    zq = _sc_gather()(embedding, idx).reshape(B, L, D)
    vq_loss = jnp.sum(minval) * (1.25 / (B * L * D))
    return zq, z_imag, vq_loss
